# baseline (speedup 1.0000x reference)
import jax
import jax.numpy as jnp
from jax import lax
from jax.experimental import pallas as pl
from jax.experimental.pallas import tpu as pltpu
from jax.experimental.pallas import tpu_sc as plsc

BATCH = 4096
MAX_LANDMARKS = 50
EMBED_DIM = 128

NC = 2
NS = 16
NW = NC * NS
L = 16

BPW = BATCH // NW
NBUF = 4


def _rsqrt16(s):
    f = s * 0.0 + 1.0
    for e in (64, 32, 16, 8, 4, 2):
        big = s >= 2.0 ** e
        s = jnp.where(big, s * 2.0 ** -e, s)
        f = jnp.where(big, f * 2.0 ** (-e // 2), f)
        small = s < 4.0 * 2.0 ** -e
        s = jnp.where(small, s * 2.0 ** e, s)
        f = jnp.where(small, f * 2.0 ** (e // 2), f)
    y = 7.0 / 6.0 - s * (1.0 / 6.0)
    for _ in range(4):
        y = y * (1.5 - 0.5 * s * y * y)
    return y * f


def _normalize_item(buf):
    iota = lax.iota(jnp.int32, L)

    def row_sumsq(row):
        acc = None
        for k in range(8):
            c = buf[row, pl.ds(L * k, L)]
            acc = c * c if acc is None else acc + c * c
        for sh in (8, 4, 2, 1):
            acc = acc + acc.at[iota ^ sh].get(mode="promise_in_bounds")
        return acc

    def row_scale(row, inv, lane):
        b = inv.at[iota * 0 + lane].get(mode="promise_in_bounds")
        for k in range(8):
            buf[row, pl.ds(L * k, L)] = buf[row, pl.ds(L * k, L)] * b

    def group(g, _):
        def quad1(q, tot):
            for r in range(4):
                lane = q * 4 + r
                tot = jnp.where(iota == lane, row_sumsq(g * L + lane), tot)
            return tot

        tot = lax.fori_loop(0, 4, quad1, jnp.zeros((L,), jnp.float32))
        inv = _rsqrt16(tot)

        def quad2(q, _):
            for r in range(4):
                lane = q * 4 + r
                row_scale(g * L + lane, inv, lane)
            return 0

        lax.fori_loop(0, 4, quad2, 0)
        return 0

    lax.fori_loop(0, MAX_LANDMARKS // L, group, 0)

    t0 = MAX_LANDMARKS - MAX_LANDMARKS % L
    tot = jnp.zeros((L,), jnp.float32)
    for j in range(MAX_LANDMARKS % L):
        tot = jnp.where(iota == j, row_sumsq(t0 + j), tot)
    inv = _rsqrt16(tot)
    for j in range(MAX_LANDMARKS % L):
        row_scale(t0 + j, inv, j)


def _gather_body(idx_hbm, table_hbm, out_hbm, idx_v, *scr):
    rows = scr[:NBUF]
    gsem = scr[NBUF:2 * NBUF]
    osem = scr[2 * NBUF:3 * NBUF]
    wid = lax.axis_index("s") * NC + lax.axis_index("c")
    base = wid * BPW
    pltpu.sync_copy(idx_hbm.at[pl.ds(base, BPW)], idx_v)

    def gfire(i, b):
        pltpu.async_copy(table_hbm.at[idx_v.at[i]], rows[b], gsem[b])

    def gwait(i, b):
        pltpu.make_async_copy(table_hbm.at[idx_v.at[i]], rows[b], gsem[b]).wait()

    def ofire(i, b):
        pltpu.async_copy(rows[b], out_hbm.at[base + i], osem[b])

    def owait(i, b):
        pltpu.make_async_copy(rows[b], out_hbm.at[base + i], osem[b]).wait()

    def visit(i, b, b2, refill, drain):
        gwait(i, b)
        _normalize_item(rows[b])
        ofire(i, b)
        if drain:
            owait(i - 2, b2)
        if refill:
            gfire(i + 2, b2)

    gfire(0, 0)
    gfire(1, 1)
    visit(0, 0, 2, True, False)
    visit(1, 1, 3, True, False)

    def round_(m, _):
        v0 = 2 + 4 * m
        for j in range(4):
            visit(v0 + j, (2 + j) % NBUF, j % NBUF, True, True)
        return 0

    lax.fori_loop(0, (BPW - NBUF) // NBUF, round_, 0)

    visit(BPW - 2, 2, 0, False, True)
    visit(BPW - 1, 3, 1, False, True)
    owait(BPW - 2, 2)
    owait(BPW - 1, 3)


def _sc_gather_normalize(indices, table):
    scratch = (
        [pltpu.VMEM((BPW, MAX_LANDMARKS), jnp.int32)]
        + [pltpu.VMEM((MAX_LANDMARKS, EMBED_DIM), jnp.float32) for _ in range(NBUF)]
        + [pltpu.SemaphoreType.DMA for _ in range(2 * NBUF)]
    )
    return pl.kernel(
        _gather_body,
        out_type=jax.ShapeDtypeStruct((BATCH, MAX_LANDMARKS, EMBED_DIM), jnp.float32),
        mesh=plsc.VectorSubcoreMesh(core_axis_name="c", subcore_axis_name="s"),
        scratch_types=scratch,
    )(indices, table)


@jax.jit
def _run(indices, table):
    features = _sc_gather_normalize(indices, table)
    mask = jnp.zeros(indices.shape, dtype=bool)
    return features, mask


def kernel(indices, table):
    return _run(indices, table)


# speedup vs baseline: 1.0742x; 1.0742x over previous
import jax
import jax.numpy as jnp
from jax import lax
from jax.experimental import pallas as pl
from jax.experimental.pallas import tpu as pltpu
from jax.experimental.pallas import tpu_sc as plsc

BATCH = 4096
MAX_LANDMARKS = 50
EMBED_DIM = 128

NC = 2
NS = 16
NW = NC * NS
L = 16

BPW = BATCH // NW
NBUF = 6
DEPTH = 3
ROWS_PAD = 64


def _rsqrt16(s):
    f = s * 0.0 + 1.0
    for e in (64, 32, 16, 8, 4, 2):
        big = s >= 2.0 ** e
        s = jnp.where(big, s * 2.0 ** -e, s)
        f = jnp.where(big, f * 2.0 ** (-e // 2), f)
        small = s < 4.0 * 2.0 ** -e
        s = jnp.where(small, s * 2.0 ** e, s)
        f = jnp.where(small, f * 2.0 ** (e // 2), f)
    y = 7.0 / 6.0 - s * (1.0 / 6.0)
    for _ in range(4):
        y = y * (1.5 - 0.5 * s * y * y)
    return y * f


def _normalize_item(buf):
    iota = lax.iota(jnp.int32, L)

    def group(g, _):
        def quad1(q, tot):
            for r in range(4):
                lane = q * 4 + r
                row = g * L + lane
                acc = None
                for k in range(8):
                    c = buf[row, pl.ds(L * k, L)]
                    acc = c * c if acc is None else acc + c * c
                for sh in (8, 4, 2, 1):
                    acc = acc + acc.at[iota ^ sh].get(mode="promise_in_bounds")
                tot = jnp.where(iota == lane, acc, tot)
            return tot

        tot = lax.fori_loop(0, 4, quad1, jnp.zeros((L,), jnp.float32))
        inv = _rsqrt16(tot)

        def quad2(q, _):
            for r in range(4):
                lane = q * 4 + r
                row = g * L + lane
                b = inv.at[iota * 0 + lane].get(mode="promise_in_bounds")
                for k in range(8):
                    buf[row, pl.ds(L * k, L)] = buf[row, pl.ds(L * k, L)] * b
            return 0

        lax.fori_loop(0, 4, quad2, 0)
        return 0

    lax.fori_loop(0, 4, group, 0)


def _gather_body(idx_hbm, table_hbm, out_hbm, idx_v, *scr):
    rows = scr[:NBUF]
    gsem = scr[NBUF:2 * NBUF]
    osem = scr[2 * NBUF:3 * NBUF]
    wid = lax.axis_index("s") * NC + lax.axis_index("c")
    base = wid * BPW
    pltpu.sync_copy(idx_hbm.at[pl.ds(base, BPW)], idx_v)

    def gfire(i, b):
        pltpu.async_copy(
            table_hbm.at[idx_v.at[i]], rows[b].at[pl.ds(0, MAX_LANDMARKS)], gsem[b])

    def gwait(i, b):
        pltpu.make_async_copy(
            table_hbm.at[idx_v.at[i]], rows[b].at[pl.ds(0, MAX_LANDMARKS)],
            gsem[b]).wait()

    def ofire(i, b):
        pltpu.async_copy(
            rows[b].at[pl.ds(0, MAX_LANDMARKS)], out_hbm.at[base + i], osem[b])

    def owait(i, b):
        pltpu.make_async_copy(
            rows[b].at[pl.ds(0, MAX_LANDMARKS)], out_hbm.at[base + i],
            osem[b]).wait()

    def visit(i, b, b2, refill, drain):
        gwait(i, b)
        _normalize_item(rows[b])
        ofire(i, b)
        if drain:
            owait(i - DEPTH, b2)
        if refill:
            gfire(i + DEPTH, b2)

    for b in range(DEPTH):
        gfire(b, b)
    for v in range(DEPTH):
        visit(v, v % NBUF, (v + DEPTH) % NBUF, True, False)

    refill_visits = BPW - 2 * DEPTH
    rounds = refill_visits // NBUF
    tail_refill = refill_visits % NBUF

    def round_(m, _):
        v0 = DEPTH + NBUF * m
        for j in range(NBUF):
            i = v0 + j
            visit(i, (DEPTH + j) % NBUF, (2 * DEPTH + j) % NBUF, True, True)
        return 0

    lax.fori_loop(0, rounds, round_, 0)

    for t in range(tail_refill):
        i = DEPTH + NBUF * rounds + t
        visit(i, i % NBUF, (i + DEPTH) % NBUF, True, True)
    for i in range(BPW - DEPTH, BPW):
        visit(i, i % NBUF, (i + DEPTH) % NBUF, False, True)
    for i in range(BPW - DEPTH, BPW):
        owait(i, i % NBUF)


def _sc_gather_normalize(indices, table):
    scratch = (
        [pltpu.VMEM((BPW, MAX_LANDMARKS), jnp.int32)]
        + [pltpu.VMEM((ROWS_PAD, EMBED_DIM), jnp.float32) for _ in range(NBUF)]
        + [pltpu.SemaphoreType.DMA for _ in range(2 * NBUF)]
    )
    return pl.kernel(
        _gather_body,
        out_type=jax.ShapeDtypeStruct((BATCH, MAX_LANDMARKS, EMBED_DIM), jnp.float32),
        mesh=plsc.VectorSubcoreMesh(core_axis_name="c", subcore_axis_name="s"),
        scratch_types=scratch,
    )(indices, table)


@jax.jit
def _run(indices, table):
    features = _sc_gather_normalize(indices, table)
    mask = jnp.zeros(indices.shape, dtype=bool)
    return features, mask


def kernel(indices, table):
    return _run(indices, table)
